# baseline (speedup 1.0000x reference)
import functools
import jax
import jax.numpy as jnp
from jax import lax
from jax.experimental import pallas as pl
from jax.experimental.pallas import tpu as pltpu
from jax.experimental.pallas import tpu_sc as plsc

_P = 0.1
_BLK = 32
_HALF = 256


def _delete_mask_const(B, L, dtype=jnp.float32):
    rand = jax.random.uniform(jax.random.key(42), (B, L))
    return (rand < _P).astype(dtype)


def _mask_kernel(msg_ref, rand_ref, idx_ref):
    g = pl.program_id(0)
    for r in range(msg_ref.shape[0]):
        _mask_one_row(msg_ref, rand_ref, idx_ref, g, r)


def _mask_one_row(msg_ref, rand_ref, idx_ref, g, r):
    b = g * msg_ref.shape[0] + r
    m = msg_ref[r]
    L, V = m.shape
    f32, i32 = jnp.float32, jnp.int32

    col0 = m[:, 0:1]
    rmax = jnp.max(m, axis=1, keepdims=True)
    nz_col = (rmax > col0).astype(f32)

    iota_col = lax.broadcasted_iota(i32, (L, 1), 0).astype(f32)
    iota_row = lax.broadcasted_iota(i32, (1, L), 1).astype(f32)
    eye = (lax.broadcasted_iota(i32, (L, L), 0) ==
           lax.broadcasted_iota(i32, (L, L), 1)).astype(f32)

    nz_row = lax.dot_general(nz_col, eye, (((0,), (0,)), ((), ())),
                             preferred_element_type=f32)
    randlt = rand_ref[r]
    keep_row = 1.0 - nz_row * randlt

    tri = (lax.broadcasted_iota(i32, (L, L), 0) <=
           lax.broadcasted_iota(i32, (L, L), 1)).astype(f32)
    prefix = jnp.dot(keep_row, tri, preferred_element_type=f32)
    kc = jnp.sum(keep_row)
    dest = prefix - 1.0

    sel = (iota_col == dest).astype(f32) * keep_row
    src = lax.dot_general(iota_row, sel, (((1,), (1,)), ((), ())),
                          preferred_element_type=f32)
    pad = (iota_row >= kc).astype(f32)
    idx_ref[r] = (src + pad * 65536.0).astype(i32) + b * L


def _sc_gather_body(table, idx_hbm, eos_hbm, out_hbm,
                    idx_v, idx_g, rows_v, eos_v,
                    gsem0, gsem1, wsem0, wsem1):
    L = 512
    i32 = jnp.int32
    c = lax.axis_index("c")
    s = lax.axis_index("s")
    wid = c * 16 + s
    b = wid // 2
    half = wid % 2
    g0 = b * L + half * _HALF

    icp = pltpu.async_copy(idx_hbm.at[pl.ds(g0, _HALF)], idx_v, gsem0)
    ecp = pltpu.async_copy(eos_hbm, eos_v, wsem0)
    icp.wait()

    for k in range(_HALF // 16):
        iv = jnp.bitwise_and(idx_v[pl.ds(k * 16, 16)], 65535)
        idx_g[k // (_BLK // 16), pl.ds((k % (_BLK // 16)) * 16, 16)] = iv

    nblk = _HALF // _BLK
    gsems = (gsem0, gsem1)
    wsems = (wsem0, wsem1)
    lanes = lax.broadcasted_iota(i32, (16,), 0)

    info = []
    for bi in range(nblk):
        m0 = idx_v[pl.ds(bi * _BLK, 16)] >= 65536
        m1 = idx_v[pl.ds(bi * _BLK + 16, 16)] >= 65536
        allf = jnp.logical_and(jnp.all(m0), jnp.all(m1))
        anyf = jnp.logical_or(jnp.any(m0), jnp.any(m1))
        info.append((m0, m1, allf, anyf))

    def gather_args(bi):
        return (table.at[idx_g.at[bi]], rows_v.at[bi % 2], gsems[bi % 2])

    def start_gather(bi):
        @pl.when(jnp.logical_not(info[bi][2]))
        def _():
            pltpu.async_copy(*gather_args(bi))

    def wait_gather(bi):
        @pl.when(jnp.logical_not(info[bi][2]))
        def _():
            pltpu.make_async_copy(*gather_args(bi)).wait()

    ecp.wait()
    wr = {}
    start_gather(0)
    for bi in range(nblk):
        if bi + 1 < nblk:
            if bi - 1 >= 0:
                wr.pop(bi - 1).wait()
            start_gather(bi + 1)
        wait_gather(bi)
        m0, m1, allf, anyf = info[bi]
        dst = out_hbm.at[pl.ds(g0 + bi * _BLK, _BLK)]

        @pl.when(jnp.logical_and(anyf, jnp.logical_not(allf)))
        def _(bi=bi, m0=m0, m1=m1):
            one16 = (lanes == 0).astype(jnp.float32)
            z16 = jnp.zeros((16,), jnp.float32)

            def row_body(r, carry):
                mk = jnp.where(r < 16, m0, m1)
                is_eos = jnp.any(jnp.logical_and(mk, lanes == r % 16))

                @pl.when(is_eos)
                def _():
                    rows_v[bi % 2, r, pl.ds(0, 16)] = one16
                    for k in range(1, 64):
                        rows_v[bi % 2, r, pl.ds(k * 16, 16)] = z16
                return carry
            lax.fori_loop(0, _BLK, row_body, 0)

        @pl.when(jnp.logical_not(allf))
        def _():
            pltpu.async_copy(rows_v.at[bi % 2], dst, wsems[bi % 2])

        @pl.when(allf)
        def _():
            pltpu.async_copy(eos_v,
                             out_hbm.at[pl.ds(g0 + bi * _BLK, 16)],
                             wsems[bi % 2])
            pltpu.async_copy(eos_v,
                             out_hbm.at[pl.ds(g0 + bi * _BLK + 16, 16)],
                             wsems[bi % 2])

        wr[bi] = pltpu.make_async_copy(rows_v.at[bi % 2], dst,
                                       wsems[bi % 2])
    for bi in sorted(wr):
        wr.pop(bi).wait()


def kernel(message, message_length, apply_noise):
    del message_length
    B, L, V = message.shape
    f32, i32 = jnp.float32, jnp.int32
    randlt = _delete_mask_const(B, L).reshape(B, 1, L)

    RB = 4
    idx_out = pl.pallas_call(
        _mask_kernel,
        grid=(B // RB,),
        in_specs=[
            pl.BlockSpec((RB, L, V), lambda b: (b, 0, 0)),
            pl.BlockSpec((RB, 1, L), lambda b: (b, 0, 0)),
        ],
        out_specs=pl.BlockSpec((RB, 1, L), lambda b: (b, 0, 0)),
        out_shape=jax.ShapeDtypeStruct((B, 1, L), i32),
        compiler_params=pltpu.CompilerParams(
            dimension_semantics=("arbitrary",),
        ),
    )(message, randlt)

    idx_flat = idx_out.reshape(B * L)
    table = message.reshape(B * L, V)
    eos = jnp.zeros((16, V), f32).at[:, 0].set(1.0)

    mesh = plsc.VectorSubcoreMesh(core_axis_name="c", subcore_axis_name="s")
    sc_call = functools.partial(
        pl.kernel, mesh=mesh,
        compiler_params=pltpu.CompilerParams(needs_layout_passes=False,
                                             skip_device_barrier=True),
        out_type=jax.ShapeDtypeStruct((B * L, V), f32),
        scratch_types=[
            pltpu.VMEM((_HALF,), i32),
            pltpu.VMEM((_HALF // _BLK, _BLK), i32),
            pltpu.VMEM((2, _BLK, V), f32),
            pltpu.VMEM((16, V), f32),
            pltpu.SemaphoreType.DMA,
            pltpu.SemaphoreType.DMA,
            pltpu.SemaphoreType.DMA,
            pltpu.SemaphoreType.DMA,
        ],
    )
    out_flat = sc_call(_sc_gather_body)(table, idx_flat, eos)

    out = out_flat.reshape(B, L, V)
    return jnp.where(jnp.asarray(apply_noise) != 0, out, message)


# speedup vs baseline: 1.0647x; 1.0647x over previous
import functools
import jax
import jax.numpy as jnp
from jax import lax
from jax.experimental import pallas as pl
from jax.experimental.pallas import tpu as pltpu
from jax.experimental.pallas import tpu_sc as plsc

_P = 0.1
_BLK = 32
_HALF = 256


def _delete_mask_const(B, L, dtype=jnp.float32):
    rand = jax.random.uniform(jax.random.key(42), (B, L))
    return (rand < _P).astype(dtype)


def _mask_kernel(msg_ref, rand_ref, idx_ref):
    g = pl.program_id(0)
    for r in range(msg_ref.shape[0]):
        _mask_one_row(msg_ref, rand_ref, idx_ref, g, r)


def _mask_one_row(msg_ref, rand_ref, idx_ref, g, r):
    b = g * msg_ref.shape[0] + r
    m = msg_ref[r]
    L, V = m.shape
    f32, i32 = jnp.float32, jnp.int32

    col0 = m[:, 0:1]
    rmax = jnp.max(m, axis=1, keepdims=True)
    nz_col = (rmax > col0).astype(f32)

    iota_col = lax.broadcasted_iota(i32, (L, 1), 0).astype(f32)
    iota_row = lax.broadcasted_iota(i32, (1, L), 1).astype(f32)
    eye = (lax.broadcasted_iota(i32, (L, L), 0) ==
           lax.broadcasted_iota(i32, (L, L), 1)).astype(f32)

    nz_row = lax.dot_general(nz_col, eye, (((0,), (0,)), ((), ())),
                             preferred_element_type=f32)
    randlt = rand_ref[r]
    keep_row = 1.0 - nz_row * randlt

    tri = (lax.broadcasted_iota(i32, (L, L), 0) <=
           lax.broadcasted_iota(i32, (L, L), 1)).astype(f32)
    prefix = jnp.dot(keep_row, tri, preferred_element_type=f32)
    kc = jnp.sum(keep_row)
    dest = prefix - 1.0

    sel = (iota_col == dest).astype(f32) * keep_row
    src = lax.dot_general(iota_row, sel, (((1,), (1,)), ((), ())),
                          preferred_element_type=f32)
    pad = (iota_row >= kc).astype(f32)
    idx_ref[r] = (src + pad * 65536.0).astype(i32) + b * L


def _sc_gather_body(table, idx_hbm, out_hbm,
                    idx_v, idx_g, rows_v, eos_v,
                    gsem0, gsem1, wsem0, wsem1):
    L = 512
    i32 = jnp.int32
    c = lax.axis_index("c")
    s = lax.axis_index("s")
    wid = c * 16 + s
    b = wid // 2
    half = wid % 2
    g0 = b * L + half * _HALF

    icp = pltpu.async_copy(idx_hbm.at[pl.ds(g0, _HALF)], idx_v, gsem0)

    nblk = _HALF // _BLK
    gsems = (gsem0, gsem1)
    wsems = (wsem0, wsem1)
    lanes = lax.broadcasted_iota(i32, (16,), 0)

    one16_f = (lanes == 0).astype(jnp.float32)
    z16_f = jnp.zeros((16,), jnp.float32)

    def eos_fill(r, carry):
        eos_v[r, pl.ds(0, 16)] = one16_f
        for k in range(1, 64):
            eos_v[r, pl.ds(k * 16, 16)] = z16_f
        return carry
    lax.fori_loop(0, 16, eos_fill, 0)

    icp.wait()

    for k in range(_HALF // 16):
        iv = jnp.bitwise_and(idx_v[pl.ds(k * 16, 16)], 65535)
        idx_g[k // (_BLK // 16), pl.ds((k % (_BLK // 16)) * 16, 16)] = iv

    info = []
    for bi in range(nblk):
        m0 = idx_v[pl.ds(bi * _BLK, 16)] >= 65536
        m1 = idx_v[pl.ds(bi * _BLK + 16, 16)] >= 65536
        allf = jnp.logical_and(jnp.all(m0), jnp.all(m1))
        anyf = jnp.logical_or(jnp.any(m0), jnp.any(m1))
        info.append((m0, m1, allf, anyf))

    def gather_args(bi):
        return (table.at[idx_g.at[bi]], rows_v.at[bi % 2], gsems[bi % 2])

    def start_gather(bi):
        @pl.when(jnp.logical_not(info[bi][2]))
        def _():
            pltpu.async_copy(*gather_args(bi))

    def wait_gather(bi):
        @pl.when(jnp.logical_not(info[bi][2]))
        def _():
            pltpu.make_async_copy(*gather_args(bi)).wait()

    wr = {}
    start_gather(0)
    for bi in range(nblk):
        if bi + 1 < nblk:
            if bi - 1 >= 0:
                wr.pop(bi - 1).wait()
            start_gather(bi + 1)
        wait_gather(bi)
        m0, m1, allf, anyf = info[bi]
        dst = out_hbm.at[pl.ds(g0 + bi * _BLK, _BLK)]

        @pl.when(jnp.logical_and(anyf, jnp.logical_not(allf)))
        def _(bi=bi, m0=m0, m1=m1):
            one16 = (lanes == 0).astype(jnp.float32)
            z16 = jnp.zeros((16,), jnp.float32)

            def row_body(r, carry):
                mk = jnp.where(r < 16, m0, m1)
                is_eos = jnp.any(jnp.logical_and(mk, lanes == r % 16))

                @pl.when(is_eos)
                def _():
                    rows_v[bi % 2, r, pl.ds(0, 16)] = one16
                    for k in range(1, 64):
                        rows_v[bi % 2, r, pl.ds(k * 16, 16)] = z16
                return carry
            lax.fori_loop(0, _BLK, row_body, 0)

        @pl.when(jnp.logical_not(allf))
        def _():
            pltpu.async_copy(rows_v.at[bi % 2], dst, wsems[bi % 2])

        @pl.when(allf)
        def _():
            pltpu.async_copy(eos_v,
                             out_hbm.at[pl.ds(g0 + bi * _BLK, 16)],
                             wsems[bi % 2])
            pltpu.async_copy(eos_v,
                             out_hbm.at[pl.ds(g0 + bi * _BLK + 16, 16)],
                             wsems[bi % 2])

        wr[bi] = pltpu.make_async_copy(rows_v.at[bi % 2], dst,
                                       wsems[bi % 2])
    for bi in sorted(wr):
        wr.pop(bi).wait()


def kernel(message, message_length, apply_noise):
    del message_length
    B, L, V = message.shape
    f32, i32 = jnp.float32, jnp.int32
    randlt = _delete_mask_const(B, L).reshape(B, 1, L)

    RB = 4
    idx_out = pl.pallas_call(
        _mask_kernel,
        grid=(B // RB,),
        in_specs=[
            pl.BlockSpec((RB, L, V), lambda b: (b, 0, 0)),
            pl.BlockSpec((RB, 1, L), lambda b: (b, 0, 0)),
        ],
        out_specs=pl.BlockSpec((RB, 1, L), lambda b: (b, 0, 0)),
        out_shape=jax.ShapeDtypeStruct((B, 1, L), i32),
        compiler_params=pltpu.CompilerParams(
            dimension_semantics=("arbitrary",),
        ),
    )(message, randlt)

    idx_flat = idx_out.reshape(B * L)
    table = message.reshape(B * L, V)

    mesh = plsc.VectorSubcoreMesh(core_axis_name="c", subcore_axis_name="s")
    sc_call = functools.partial(
        pl.kernel, mesh=mesh,
        compiler_params=pltpu.CompilerParams(needs_layout_passes=False,
                                             skip_device_barrier=True),
        out_type=jax.ShapeDtypeStruct((B * L, V), f32),
        scratch_types=[
            pltpu.VMEM((_HALF,), i32),
            pltpu.VMEM((_HALF // _BLK, _BLK), i32),
            pltpu.VMEM((2, _BLK, V), f32),
            pltpu.VMEM((16, V), f32),
            pltpu.SemaphoreType.DMA,
            pltpu.SemaphoreType.DMA,
            pltpu.SemaphoreType.DMA,
            pltpu.SemaphoreType.DMA,
        ],
    )
    out_flat = sc_call(_sc_gather_body)(table, idx_flat)

    out = out_flat.reshape(B, L, V)
    return jnp.where(jnp.asarray(apply_noise) != 0, out, message)
